# baseline (speedup 1.0000x reference)
import functools

import jax
import jax.numpy as jnp
from jax import lax
from jax.experimental import pallas as pl
from jax.experimental.pallas import tpu as pltpu
from jax.experimental.pallas import tpu_sc as plsc

NUM_LEVELS = 16
FEATS = 2
HASHMAP_SIZE = 1 << 19
BASE_RES = 16

N_POINTS = 8 * 65536
NUM_WORKERS = 32
PTS_PER_W = N_POINTS // NUM_WORKERS
CH = 128
NCHUNK = PTS_PER_W // CH
G = CH // 16
OUT_F = NUM_LEVELS * FEATS
PACK = 4
ROW_F = PACK * FEATS

_mesh = plsc.VectorSubcoreMesh(core_axis_name="c", subcore_axis_name="s")


@functools.partial(
    pl.kernel,
    out_type=jax.ShapeDtypeStruct((N_POINTS * OUT_F,), jnp.float32),
    mesh=_mesh,
    scratch_types=[
        pltpu.VMEM((CH,), jnp.float32),
        pltpu.VMEM((CH,), jnp.float32),
        pltpu.VMEM((NUM_LEVELS * CH,), jnp.float32),
        pltpu.VMEM((NUM_LEVELS * CH,), jnp.float32),
        pltpu.VMEM((NUM_LEVELS * 4, CH), jnp.int32),
        pltpu.VMEM((NUM_LEVELS * 4, CH), jnp.int32),
        pltpu.VMEM((NUM_LEVELS * 4, CH, ROW_F), jnp.float32),
        pltpu.VMEM((CH * OUT_F,), jnp.float32),
        pltpu.SemaphoreType.DMA,
    ],
    compiler_params=pltpu.CompilerParams(needs_layout_passes=False,
                                         use_tc_tiling_on_sc=False),
)
def _encode(u_hbm, v_hbm, tab_hbm, out_hbm,
            u_v, v_v, fx_b, fy_b, idx_b, low_b, rows_b, out_b, gsem):
    wid = lax.axis_index("s") * 2 + lax.axis_index("c")
    base = wid * PTS_PER_W
    iota = lax.iota(jnp.int32, 16)
    iota32 = iota * OUT_F
    mask = HASHMAP_SIZE - 1

    @pl.loop(0, NCHUNK)
    def _chunk(c):
        off = base + c * CH
        pltpu.sync_copy(u_hbm.at[pl.ds(off, CH)], u_v)
        pltpu.sync_copy(v_hbm.at[pl.ds(off, CH)], v_v)

        copies = []
        for lvl in range(NUM_LEVELS):
            res = BASE_RES * (2 ** lvl)
            rm1 = float(res - 1)
            lvl_off4 = lvl * (HASHMAP_SIZE // PACK)
            j = lvl * 4

            @pl.loop(0, G)
            def _build(g):
                sl = pl.ds(g * 16, 16)
                fsl = pl.ds(lvl * CH + g * 16, 16)
                u = u_v[sl]
                v = v_v[sl]
                gx = (u + 1.0) * 0.5 * rm1
                gy = (v + 1.0) * 0.5 * rm1
                ix = jnp.minimum(jnp.maximum(gx.astype(jnp.int32), 0), res - 1)
                iy = jnp.minimum(jnp.maximum(gy.astype(jnp.int32), 0), res - 1)
                fx_b[fsl] = gx - ix.astype(jnp.float32)
                fy_b[fsl] = gy - iy.astype(jnp.float32)
                hx0 = ix * 73856093
                hx1 = hx0 + 73856093
                hy0 = iy * 19349663
                hy1 = hy0 + 19349663
                h00 = (hx0 ^ hy0) & mask
                h10 = (hx1 ^ hy0) & mask
                h01 = (hx0 ^ hy1) & mask
                h11 = (hx1 ^ hy1) & mask
                idx_b[j + 0, sl] = (h00 >> 2) + lvl_off4
                idx_b[j + 1, sl] = (h10 >> 2) + lvl_off4
                idx_b[j + 2, sl] = (h01 >> 2) + lvl_off4
                idx_b[j + 3, sl] = (h11 >> 2) + lvl_off4
                low_b[j + 0, sl] = (h00 & 3) * 2
                low_b[j + 1, sl] = (h10 & 3) * 2
                low_b[j + 2, sl] = (h01 & 3) * 2
                low_b[j + 3, sl] = (h11 & 3) * 2

            for cc in range(4):
                copies.append(
                    pltpu.async_copy(tab_hbm.at[idx_b.at[j + cc]],
                                     rows_b.at[j + cc], gsem))

        for d in copies:
            d.wait()

        for lvl in range(NUM_LEVELS):
            j = lvl * 4

            @pl.loop(0, G)
            def _combine(g):
                sl = pl.ds(g * 16, 16)
                fsl = pl.ds(lvl * CH + g * 16, 16)
                fx = fx_b[fsl]
                fy = fy_b[fsl]
                wx0 = 1.0 - fx
                wy0 = 1.0 - fy
                w00 = wx0 * wy0
                w10 = fx * wy0
                w01 = wx0 * fy
                w11 = fx * fy
                pt = g * 16 + iota
                c00 = low_b[j + 0, sl]
                c10 = low_b[j + 1, sl]
                c01 = low_b[j + 2, sl]
                c11 = low_b[j + 3, sl]
                e00a = plsc.load_gather(rows_b.at[j + 0], [pt, c00])
                e00b = plsc.load_gather(rows_b.at[j + 0], [pt, c00 + 1])
                e10a = plsc.load_gather(rows_b.at[j + 1], [pt, c10])
                e10b = plsc.load_gather(rows_b.at[j + 1], [pt, c10 + 1])
                e01a = plsc.load_gather(rows_b.at[j + 2], [pt, c01])
                e01b = plsc.load_gather(rows_b.at[j + 2], [pt, c01 + 1])
                e11a = plsc.load_gather(rows_b.at[j + 3], [pt, c11])
                e11b = plsc.load_gather(rows_b.at[j + 3], [pt, c11 + 1])
                feat0 = ((e00a * w00 + e10a * w10) + e01a * w01) + e11a * w11
                feat1 = ((e00b * w00 + e10b * w10) + e01b * w01) + e11b * w11
                obase = g * (16 * OUT_F) + 2 * lvl
                plsc.store_scatter(out_b, [iota32 + obase], feat0)
                plsc.store_scatter(out_b, [iota32 + (obase + 1)], feat1)

        pltpu.sync_copy(out_b, out_hbm.at[pl.ds(off * OUT_F, CH * OUT_F)])


def kernel(coords, tables):
    b, n, _ = coords.shape
    u = coords[..., 1].reshape(N_POINTS)
    v = coords[..., 0].reshape(N_POINTS)
    tab = tables.reshape(NUM_LEVELS * HASHMAP_SIZE // PACK, ROW_F)
    out = _encode(u, v, tab)
    return out.reshape(b, n, OUT_F)


# speedup vs baseline: 1.0499x; 1.0499x over previous
import functools

import jax
import jax.numpy as jnp
from jax import lax
from jax.experimental import pallas as pl
from jax.experimental.pallas import tpu as pltpu
from jax.experimental.pallas import tpu_sc as plsc

NUM_LEVELS = 16
FEATS = 2
HASHMAP_SIZE = 1 << 19
BASE_RES = 16

N_POINTS = 8 * 65536
NUM_WORKERS = 32
PTS_PER_W = N_POINTS // NUM_WORKERS
CH = 512
NCHUNK = PTS_PER_W // CH
G = CH // 16
OUT_F = NUM_LEVELS * FEATS
PACK = 4
ROW_F = PACK * FEATS
NIDX = 4 * CH
DEPTH = 4

_mesh = plsc.VectorSubcoreMesh(core_axis_name="c", subcore_axis_name="s")


@functools.partial(
    pl.kernel,
    out_type=jax.ShapeDtypeStruct((N_POINTS * OUT_F,), jnp.float32),
    mesh=_mesh,
    scratch_types=[
        pltpu.VMEM((CH,), jnp.float32),
        pltpu.VMEM((CH,), jnp.float32),
        pltpu.VMEM((DEPTH, CH), jnp.float32),
        pltpu.VMEM((DEPTH, CH), jnp.float32),
        pltpu.VMEM((DEPTH, NIDX), jnp.int32),
        pltpu.VMEM((DEPTH, NIDX), jnp.int32),
        pltpu.VMEM((DEPTH, NIDX, ROW_F), jnp.float32),
        pltpu.VMEM((CH * OUT_F,), jnp.float32),
        pltpu.SemaphoreType.DMA,
    ],
    compiler_params=pltpu.CompilerParams(needs_layout_passes=False,
                                         use_tc_tiling_on_sc=False),
)
def _encode(u_hbm, v_hbm, tab_hbm, out_hbm,
            u_v, v_v, fx_b, fy_b, idx_b, low_b, rows_b, out_b, gsem):
    wid = lax.axis_index("s") * 2 + lax.axis_index("c")
    base = wid * PTS_PER_W
    iota = lax.iota(jnp.int32, 16)
    iota32 = iota * OUT_F
    mask = HASHMAP_SIZE - 1

    def build(lvl):
        s = lvl % DEPTH
        res = BASE_RES * (2 ** lvl)
        rm1 = float(res - 1)
        lvl_off4 = lvl * (HASHMAP_SIZE // PACK)

        @pl.loop(0, G)
        def _build(g):
            sl = pl.ds(g * 16, 16)
            u = u_v[sl]
            v = v_v[sl]
            gx = (u + 1.0) * 0.5 * rm1
            gy = (v + 1.0) * 0.5 * rm1
            ix = jnp.minimum(jnp.maximum(gx.astype(jnp.int32), 0), res - 1)
            iy = jnp.minimum(jnp.maximum(gy.astype(jnp.int32), 0), res - 1)
            fx_b[s, sl] = gx - ix.astype(jnp.float32)
            fy_b[s, sl] = gy - iy.astype(jnp.float32)
            hx0 = ix * 73856093
            hx1 = hx0 + 73856093
            hy0 = iy * 19349663
            hy1 = hy0 + 19349663
            h00 = (hx0 ^ hy0) & mask
            h10 = (hx1 ^ hy0) & mask
            h01 = (hx0 ^ hy1) & mask
            h11 = (hx1 ^ hy1) & mask
            idx_b[s, pl.ds(0 * CH + g * 16, 16)] = (h00 >> 2) + lvl_off4
            idx_b[s, pl.ds(1 * CH + g * 16, 16)] = (h10 >> 2) + lvl_off4
            idx_b[s, pl.ds(2 * CH + g * 16, 16)] = (h01 >> 2) + lvl_off4
            idx_b[s, pl.ds(3 * CH + g * 16, 16)] = (h11 >> 2) + lvl_off4
            low_b[s, pl.ds(0 * CH + g * 16, 16)] = (h00 & 3) * 2
            low_b[s, pl.ds(1 * CH + g * 16, 16)] = (h10 & 3) * 2
            low_b[s, pl.ds(2 * CH + g * 16, 16)] = (h01 & 3) * 2
            low_b[s, pl.ds(3 * CH + g * 16, 16)] = (h11 & 3) * 2

    def fire(lvl):
        s = lvl % DEPTH
        return pltpu.async_copy(tab_hbm.at[idx_b.at[s]], rows_b.at[s], gsem)

    def combine(lvl):
        s = lvl % DEPTH

        @pl.loop(0, G)
        def _combine(g):
            sl = pl.ds(g * 16, 16)
            fx = fx_b[s, sl]
            fy = fy_b[s, sl]
            wx0 = 1.0 - fx
            wy0 = 1.0 - fy
            w00 = wx0 * wy0
            w10 = fx * wy0
            w01 = wx0 * fy
            w11 = fx * fy
            pt = g * 16 + iota
            rows = rows_b.at[s]
            c00 = low_b[s, pl.ds(0 * CH + g * 16, 16)]
            c10 = low_b[s, pl.ds(1 * CH + g * 16, 16)]
            c01 = low_b[s, pl.ds(2 * CH + g * 16, 16)]
            c11 = low_b[s, pl.ds(3 * CH + g * 16, 16)]
            e00a = plsc.load_gather(rows, [pt, c00])
            e00b = plsc.load_gather(rows, [pt, c00 + 1])
            e10a = plsc.load_gather(rows, [CH + pt, c10])
            e10b = plsc.load_gather(rows, [CH + pt, c10 + 1])
            e01a = plsc.load_gather(rows, [2 * CH + pt, c01])
            e01b = plsc.load_gather(rows, [2 * CH + pt, c01 + 1])
            e11a = plsc.load_gather(rows, [3 * CH + pt, c11])
            e11b = plsc.load_gather(rows, [3 * CH + pt, c11 + 1])
            feat0 = ((e00a * w00 + e10a * w10) + e01a * w01) + e11a * w11
            feat1 = ((e00b * w00 + e10b * w10) + e01b * w01) + e11b * w11
            obase = g * (16 * OUT_F) + 2 * lvl
            plsc.store_scatter(out_b, [iota32 + obase], feat0)
            plsc.store_scatter(out_b, [iota32 + (obase + 1)], feat1)

    @pl.loop(0, NCHUNK)
    def _chunk(c):
        off = base + c * CH
        pltpu.sync_copy(u_hbm.at[pl.ds(off, CH)], u_v)
        pltpu.sync_copy(v_hbm.at[pl.ds(off, CH)], v_v)

        copies = {}
        for lvl in range(DEPTH):
            build(lvl)
            copies[lvl] = fire(lvl)
        for lvl in range(NUM_LEVELS):
            copies.pop(lvl).wait()
            combine(lvl)
            nxt = lvl + DEPTH
            if nxt < NUM_LEVELS:
                build(nxt)
                copies[nxt] = fire(nxt)

        pltpu.sync_copy(out_b, out_hbm.at[pl.ds(off * OUT_F, CH * OUT_F)])


def kernel(coords, tables):
    b, n, _ = coords.shape
    u = coords[..., 1].reshape(N_POINTS)
    v = coords[..., 0].reshape(N_POINTS)
    tab = tables.reshape(NUM_LEVELS * HASHMAP_SIZE // PACK, ROW_F)
    out = _encode(u, v, tab)
    return out.reshape(b, n, OUT_F)


# speedup vs baseline: 1.0672x; 1.0165x over previous
import functools

import jax
import jax.numpy as jnp
from jax import lax
from jax.experimental import pallas as pl
from jax.experimental.pallas import tpu as pltpu
from jax.experimental.pallas import tpu_sc as plsc

NUM_LEVELS = 16
FEATS = 2
HASHMAP_SIZE = 1 << 19
BASE_RES = 16

N_POINTS = 8 * 65536
NUM_WORKERS = 32
PTS_PER_W = N_POINTS // NUM_WORKERS
CH = 512
NCHUNK = PTS_PER_W // CH
G = CH // 16
OUT_F = NUM_LEVELS * FEATS
PACK = 4
ROW_F = PACK * FEATS
NIDX = 4 * CH
DEPTH = 4

_mesh = plsc.VectorSubcoreMesh(core_axis_name="c", subcore_axis_name="s")


@functools.partial(
    pl.kernel,
    out_type=jax.ShapeDtypeStruct((N_POINTS * OUT_F,), jnp.float32),
    mesh=_mesh,
    scratch_types=[
        pltpu.VMEM((CH,), jnp.float32),
        pltpu.VMEM((CH,), jnp.float32),
        pltpu.VMEM((DEPTH, CH), jnp.float32),
        pltpu.VMEM((DEPTH, CH), jnp.float32),
        pltpu.VMEM((DEPTH, NIDX), jnp.int32),
        pltpu.VMEM((DEPTH, NIDX), jnp.int32),
        pltpu.VMEM((DEPTH, NIDX, ROW_F), jnp.float32),
        pltpu.VMEM((OUT_F, CH), jnp.float32),
        pltpu.SemaphoreType.DMA,
        pltpu.SemaphoreType.DMA,
    ],
    compiler_params=pltpu.CompilerParams(needs_layout_passes=False,
                                         use_tc_tiling_on_sc=False),
)
def _encode(u_hbm, v_hbm, tab_hbm, out_hbm,
            u_v, v_v, fx_b, fy_b, idx_b, low_b, rows_b, out_b, gsem, osem):
    wid = lax.axis_index("s") * 2 + lax.axis_index("c")
    base = wid * PTS_PER_W
    iota = lax.iota(jnp.int32, 16)
    mask = HASHMAP_SIZE - 1

    def build(lvl):
        s = lvl % DEPTH
        res = BASE_RES * (2 ** lvl)
        rm1 = float(res - 1)
        lvl_off4 = lvl * (HASHMAP_SIZE // PACK)

        @pl.loop(0, G)
        def _build(g):
            sl = pl.ds(g * 16, 16)
            u = u_v[sl]
            v = v_v[sl]
            gx = (u + 1.0) * 0.5 * rm1
            gy = (v + 1.0) * 0.5 * rm1
            ix = jnp.minimum(jnp.maximum(gx.astype(jnp.int32), 0), res - 1)
            iy = jnp.minimum(jnp.maximum(gy.astype(jnp.int32), 0), res - 1)
            fx_b[s, sl] = gx - ix.astype(jnp.float32)
            fy_b[s, sl] = gy - iy.astype(jnp.float32)
            hx0 = ix * 73856093
            hx1 = hx0 + 73856093
            hy0 = iy * 19349663
            hy1 = hy0 + 19349663
            h00 = (hx0 ^ hy0) & mask
            h10 = (hx1 ^ hy0) & mask
            h01 = (hx0 ^ hy1) & mask
            h11 = (hx1 ^ hy1) & mask
            idx_b[s, pl.ds(0 * CH + g * 16, 16)] = (h00 >> 2) + lvl_off4
            idx_b[s, pl.ds(1 * CH + g * 16, 16)] = (h10 >> 2) + lvl_off4
            idx_b[s, pl.ds(2 * CH + g * 16, 16)] = (h01 >> 2) + lvl_off4
            idx_b[s, pl.ds(3 * CH + g * 16, 16)] = (h11 >> 2) + lvl_off4
            low_b[s, pl.ds(0 * CH + g * 16, 16)] = (h00 & 3) * 2
            low_b[s, pl.ds(1 * CH + g * 16, 16)] = (h10 & 3) * 2
            low_b[s, pl.ds(2 * CH + g * 16, 16)] = (h01 & 3) * 2
            low_b[s, pl.ds(3 * CH + g * 16, 16)] = (h11 & 3) * 2

    def fire(lvl):
        s = lvl % DEPTH
        return pltpu.async_copy(tab_hbm.at[idx_b.at[s]], rows_b.at[s], gsem)

    def combine(lvl):
        s = lvl % DEPTH

        @pl.loop(0, G)
        def _combine(g):
            sl = pl.ds(g * 16, 16)
            fx = fx_b[s, sl]
            fy = fy_b[s, sl]
            wx0 = 1.0 - fx
            wy0 = 1.0 - fy
            w00 = wx0 * wy0
            w10 = fx * wy0
            w01 = wx0 * fy
            w11 = fx * fy
            pt = g * 16 + iota
            rows = rows_b.at[s]
            c00 = low_b[s, pl.ds(0 * CH + g * 16, 16)]
            c10 = low_b[s, pl.ds(1 * CH + g * 16, 16)]
            c01 = low_b[s, pl.ds(2 * CH + g * 16, 16)]
            c11 = low_b[s, pl.ds(3 * CH + g * 16, 16)]
            e00a = plsc.load_gather(rows, [pt, c00])
            e00b = plsc.load_gather(rows, [pt, c00 + 1])
            e10a = plsc.load_gather(rows, [CH + pt, c10])
            e10b = plsc.load_gather(rows, [CH + pt, c10 + 1])
            e01a = plsc.load_gather(rows, [2 * CH + pt, c01])
            e01b = plsc.load_gather(rows, [2 * CH + pt, c01 + 1])
            e11a = plsc.load_gather(rows, [3 * CH + pt, c11])
            e11b = plsc.load_gather(rows, [3 * CH + pt, c11 + 1])
            feat0 = ((e00a * w00 + e10a * w10) + e01a * w01) + e11a * w11
            feat1 = ((e00b * w00 + e10b * w10) + e01b * w01) + e11b * w11
            out_b[2 * lvl, sl] = feat0
            out_b[2 * lvl + 1, sl] = feat1

    @pl.loop(0, NCHUNK)
    def _chunk(c):
        off = base + c * CH
        pltpu.sync_copy(u_hbm.at[pl.ds(off, CH)], u_v)
        pltpu.sync_copy(v_hbm.at[pl.ds(off, CH)], v_v)

        copies = {}
        for lvl in range(DEPTH):
            build(lvl)
            copies[lvl] = fire(lvl)
        for lvl in range(NUM_LEVELS):
            copies.pop(lvl).wait()
            combine(lvl)
            nxt = lvl + DEPTH
            if nxt < NUM_LEVELS:
                build(nxt)
                copies[nxt] = fire(nxt)

        bb = off // 65536
        pcol = off - bb * 65536
        obase = bb * (OUT_F * 65536) + pcol
        ocopies = [
            pltpu.async_copy(out_b.at[f],
                             out_hbm.at[pl.ds(obase + f * 65536, CH)], osem)
            for f in range(OUT_F)
        ]
        for d in ocopies:
            d.wait()


def kernel(coords, tables):
    b, n, _ = coords.shape
    u = coords[..., 1].reshape(N_POINTS)
    v = coords[..., 0].reshape(N_POINTS)
    tab = tables.reshape(NUM_LEVELS * HASHMAP_SIZE // PACK, ROW_F)
    out = _encode(u, v, tab)
    return out.reshape(b, OUT_F, n).transpose(0, 2, 1)


# speedup vs baseline: 4.6883x; 4.3930x over previous
import functools

import jax
import jax.numpy as jnp
from jax import lax
from jax.experimental import pallas as pl
from jax.experimental.pallas import tpu as pltpu
from jax.experimental.pallas import tpu_sc as plsc

NUM_LEVELS = 16
FEATS = 2
HASHMAP_SIZE = 1 << 19
BASE_RES = 16

N_POINTS = 8 * 65536
NUM_WORKERS = 32
PTS_PER_W = N_POINTS // NUM_WORKERS
CH = 256
NCHUNK = PTS_PER_W // CH
G = CH // 16
OUT_F = NUM_LEVELS * FEATS
PACK = 8
PLANE_ROWS = HASHMAP_SIZE // PACK
NIDX = 8 * CH
DEPTH = 4

_mesh = plsc.VectorSubcoreMesh(core_axis_name="c", subcore_axis_name="s")


@functools.partial(
    pl.kernel,
    out_type=jax.ShapeDtypeStruct((N_POINTS * OUT_F,), jnp.float32),
    mesh=_mesh,
    scratch_types=[
        pltpu.VMEM((CH,), jnp.float32),
        pltpu.VMEM((CH,), jnp.float32),
        pltpu.VMEM((DEPTH, CH), jnp.float32),
        pltpu.VMEM((DEPTH, CH), jnp.float32),
        pltpu.VMEM((DEPTH, NIDX), jnp.int32),
        pltpu.VMEM((DEPTH, 4 * CH), jnp.int32),
        pltpu.VMEM((DEPTH, NIDX, PACK), jnp.float32),
        pltpu.VMEM((OUT_F, CH), jnp.float32),
        pltpu.SemaphoreType.DMA,
        pltpu.SemaphoreType.DMA,
    ],
    compiler_params=pltpu.CompilerParams(needs_layout_passes=False,
                                         use_tc_tiling_on_sc=False),
)
def _encode(u_hbm, v_hbm, tab_hbm, out_hbm,
            u_v, v_v, fx_b, fy_b, idx_b, low_b, rows_b, out_b, gsem, osem):
    wid = lax.axis_index("s") * 2 + lax.axis_index("c")
    base = wid * PTS_PER_W
    iota = lax.iota(jnp.int32, 16)
    mask = HASHMAP_SIZE - 1

    def build(lvl):
        s = lvl % DEPTH
        res = BASE_RES * (2 ** lvl)
        rm1 = float(res - 1)
        p0 = (2 * lvl) * PLANE_ROWS

        @pl.loop(0, G)
        def _build(g):
            sl = pl.ds(g * 16, 16)
            u = u_v[sl]
            v = v_v[sl]
            gx = (u + 1.0) * 0.5 * rm1
            gy = (v + 1.0) * 0.5 * rm1
            ix = jnp.minimum(jnp.maximum(gx.astype(jnp.int32), 0), res - 1)
            iy = jnp.minimum(jnp.maximum(gy.astype(jnp.int32), 0), res - 1)
            fx_b[s, sl] = gx - ix.astype(jnp.float32)
            fy_b[s, sl] = gy - iy.astype(jnp.float32)
            hx0 = ix * 73856093
            hx1 = hx0 + 73856093
            hy0 = iy * 19349663
            hy1 = hy0 + 19349663
            h00 = (hx0 ^ hy0) & mask
            h10 = (hx1 ^ hy0) & mask
            h01 = (hx0 ^ hy1) & mask
            h11 = (hx1 ^ hy1) & mask
            r00 = (h00 >> 3) + p0
            r10 = (h10 >> 3) + p0
            r01 = (h01 >> 3) + p0
            r11 = (h11 >> 3) + p0
            idx_b[s, pl.ds(0 * CH + g * 16, 16)] = r00
            idx_b[s, pl.ds(1 * CH + g * 16, 16)] = r00 + PLANE_ROWS
            idx_b[s, pl.ds(2 * CH + g * 16, 16)] = r10
            idx_b[s, pl.ds(3 * CH + g * 16, 16)] = r10 + PLANE_ROWS
            idx_b[s, pl.ds(4 * CH + g * 16, 16)] = r01
            idx_b[s, pl.ds(5 * CH + g * 16, 16)] = r01 + PLANE_ROWS
            idx_b[s, pl.ds(6 * CH + g * 16, 16)] = r11
            idx_b[s, pl.ds(7 * CH + g * 16, 16)] = r11 + PLANE_ROWS
            low_b[s, pl.ds(0 * CH + g * 16, 16)] = h00 & 7
            low_b[s, pl.ds(1 * CH + g * 16, 16)] = h10 & 7
            low_b[s, pl.ds(2 * CH + g * 16, 16)] = h01 & 7
            low_b[s, pl.ds(3 * CH + g * 16, 16)] = h11 & 7

    def fire(lvl):
        s = lvl % DEPTH
        return pltpu.async_copy(tab_hbm.at[idx_b.at[s]], rows_b.at[s], gsem)

    def combine(lvl):
        s = lvl % DEPTH

        @pl.loop(0, G)
        def _combine(g):
            sl = pl.ds(g * 16, 16)
            fx = fx_b[s, sl]
            fy = fy_b[s, sl]
            wx0 = 1.0 - fx
            wy0 = 1.0 - fy
            w00 = wx0 * wy0
            w10 = fx * wy0
            w01 = wx0 * fy
            w11 = fx * fy
            pt = g * 16 + iota
            rows = rows_b.at[s]
            c00 = low_b[s, pl.ds(0 * CH + g * 16, 16)]
            c10 = low_b[s, pl.ds(1 * CH + g * 16, 16)]
            c01 = low_b[s, pl.ds(2 * CH + g * 16, 16)]
            c11 = low_b[s, pl.ds(3 * CH + g * 16, 16)]
            e00a = plsc.load_gather(rows, [0 * CH + pt, c00])
            e00b = plsc.load_gather(rows, [1 * CH + pt, c00])
            e10a = plsc.load_gather(rows, [2 * CH + pt, c10])
            e10b = plsc.load_gather(rows, [3 * CH + pt, c10])
            e01a = plsc.load_gather(rows, [4 * CH + pt, c01])
            e01b = plsc.load_gather(rows, [5 * CH + pt, c01])
            e11a = plsc.load_gather(rows, [6 * CH + pt, c11])
            e11b = plsc.load_gather(rows, [7 * CH + pt, c11])
            feat0 = ((e00a * w00 + e10a * w10) + e01a * w01) + e11a * w11
            feat1 = ((e00b * w00 + e10b * w10) + e01b * w01) + e11b * w11
            out_b[2 * lvl, sl] = feat0
            out_b[2 * lvl + 1, sl] = feat1

    @pl.loop(0, NCHUNK)
    def _chunk(c):
        off = base + c * CH
        pltpu.sync_copy(u_hbm.at[pl.ds(off, CH)], u_v)
        pltpu.sync_copy(v_hbm.at[pl.ds(off, CH)], v_v)

        copies = {}
        for lvl in range(DEPTH):
            build(lvl)
            copies[lvl] = fire(lvl)
        for lvl in range(NUM_LEVELS):
            copies.pop(lvl).wait()
            combine(lvl)
            nxt = lvl + DEPTH
            if nxt < NUM_LEVELS:
                build(nxt)
                copies[nxt] = fire(nxt)

        bb = off // 65536
        pcol = off - bb * 65536
        obase = bb * (OUT_F * 65536) + pcol
        ocopies = [
            pltpu.async_copy(out_b.at[f],
                             out_hbm.at[pl.ds(obase + f * 65536, CH)], osem)
            for f in range(OUT_F)
        ]
        for d in ocopies:
            d.wait()


def kernel(coords, tables):
    b, n, _ = coords.shape
    u = coords[..., 1].reshape(N_POINTS)
    v = coords[..., 0].reshape(N_POINTS)
    tab = tables.transpose(0, 2, 1).reshape(
        NUM_LEVELS * FEATS * PLANE_ROWS, PACK)
    out = _encode(u, v, tab)
    return out.reshape(b, OUT_F, n).transpose(0, 2, 1)


# speedup vs baseline: 7.6856x; 1.6393x over previous
import functools

import jax
import jax.numpy as jnp
from jax import lax
from jax.experimental import pallas as pl
from jax.experimental.pallas import tpu as pltpu
from jax.experimental.pallas import tpu_sc as plsc

NUM_LEVELS = 16
FEATS = 2
HASHMAP_SIZE = 1 << 19
BASE_RES = 16

N_POINTS = 8 * 65536
NUM_WORKERS = 32
PTS_PER_W = N_POINTS // NUM_WORKERS
CH = 512
NCHUNK = PTS_PER_W // CH
G = CH // 16
OUT_F = NUM_LEVELS * FEATS
PACK = 4
ROW_F = PACK * FEATS
NROWS = NUM_LEVELS * HASHMAP_SIZE // PACK
NIDX = 4 * CH
DEPTH = 4

TAB_F = NUM_LEVELS * HASHMAP_SIZE * FEATS
ENT_PER_W = NUM_LEVELS * HASHMAP_SIZE // NUM_WORKERS
PCHK = 8192
NPCHK = ENT_PER_W // PCHK

_mesh = plsc.VectorSubcoreMesh(core_axis_name="c", subcore_axis_name="s")


@functools.partial(
    pl.kernel,
    out_type=jax.ShapeDtypeStruct((TAB_F,), jnp.float32),
    mesh=_mesh,
    scratch_types=[
        pltpu.VMEM((PCHK,), jnp.float32),
        pltpu.VMEM((PCHK,), jnp.float32),
        pltpu.VMEM((2 * PCHK,), jnp.float32),
    ],
    compiler_params=pltpu.CompilerParams(needs_layout_passes=False,
                                         use_tc_tiling_on_sc=False),
)
def _pack(tab_hbm, packed_hbm, f0_v, f1_v, out_v):
    wid = lax.axis_index("s") * 2 + lax.axis_index("c")
    lvl = wid // 2
    half = wid - lvl * 2
    iota2 = lax.iota(jnp.int32, 16) * 2
    src0 = (2 * lvl) * HASHMAP_SIZE + half * (HASHMAP_SIZE // 2)
    ebase = lvl * HASHMAP_SIZE + half * (HASHMAP_SIZE // 2)

    @pl.loop(0, NPCHK)
    def _c(ck):
        pltpu.sync_copy(tab_hbm.at[pl.ds(src0 + ck * PCHK, PCHK)], f0_v)
        pltpu.sync_copy(tab_hbm.at[pl.ds(src0 + HASHMAP_SIZE + ck * PCHK, PCHK)],
                        f1_v)

        @pl.loop(0, PCHK // 16)
        def _g(k):
            a = f0_v[pl.ds(k * 16, 16)]
            b = f1_v[pl.ds(k * 16, 16)]
            plsc.store_scatter(out_v, [iota2 + k * 32], a)
            plsc.store_scatter(out_v, [iota2 + (k * 32 + 1)], b)

        pltpu.sync_copy(out_v,
                        packed_hbm.at[pl.ds(2 * (ebase + ck * PCHK), 2 * PCHK)])


@functools.partial(
    pl.kernel,
    out_type=jax.ShapeDtypeStruct((N_POINTS * OUT_F,), jnp.float32),
    mesh=_mesh,
    scratch_types=[
        pltpu.VMEM((CH,), jnp.float32),
        pltpu.VMEM((CH,), jnp.float32),
        pltpu.VMEM((DEPTH, CH), jnp.float32),
        pltpu.VMEM((DEPTH, CH), jnp.float32),
        pltpu.VMEM((DEPTH, NIDX), jnp.int32),
        pltpu.VMEM((DEPTH, NIDX), jnp.int32),
        pltpu.VMEM((DEPTH, NIDX, ROW_F), jnp.float32),
        pltpu.VMEM((OUT_F, CH), jnp.float32),
        pltpu.SemaphoreType.DMA,
        pltpu.SemaphoreType.DMA,
    ],
    compiler_params=pltpu.CompilerParams(needs_layout_passes=False,
                                         use_tc_tiling_on_sc=False),
)
def _encode(u_hbm, v_hbm, tab_hbm, out_hbm,
            u_v, v_v, fx_b, fy_b, idx_b, low_b, rows_b, out_b, gsem, osem):
    wid = lax.axis_index("s") * 2 + lax.axis_index("c")
    base = wid * PTS_PER_W
    iota = lax.iota(jnp.int32, 16)
    mask = HASHMAP_SIZE - 1

    def build(lvl):
        s = lvl % DEPTH
        res = BASE_RES * (2 ** lvl)
        rm1 = float(res - 1)
        lvl_off4 = lvl * (HASHMAP_SIZE // PACK)

        @pl.loop(0, G)
        def _build(g):
            sl = pl.ds(g * 16, 16)
            u = u_v[sl]
            v = v_v[sl]
            gx = (u + 1.0) * 0.5 * rm1
            gy = (v + 1.0) * 0.5 * rm1
            ix = jnp.minimum(jnp.maximum(gx.astype(jnp.int32), 0), res - 1)
            iy = jnp.minimum(jnp.maximum(gy.astype(jnp.int32), 0), res - 1)
            fx_b[s, sl] = gx - ix.astype(jnp.float32)
            fy_b[s, sl] = gy - iy.astype(jnp.float32)
            hx0 = ix * 73856093
            hx1 = hx0 + 73856093
            hy0 = iy * 19349663
            hy1 = hy0 + 19349663
            h00 = (hx0 ^ hy0) & mask
            h10 = (hx1 ^ hy0) & mask
            h01 = (hx0 ^ hy1) & mask
            h11 = (hx1 ^ hy1) & mask
            idx_b[s, pl.ds(0 * CH + g * 16, 16)] = (h00 >> 2) + lvl_off4
            idx_b[s, pl.ds(1 * CH + g * 16, 16)] = (h10 >> 2) + lvl_off4
            idx_b[s, pl.ds(2 * CH + g * 16, 16)] = (h01 >> 2) + lvl_off4
            idx_b[s, pl.ds(3 * CH + g * 16, 16)] = (h11 >> 2) + lvl_off4
            low_b[s, pl.ds(0 * CH + g * 16, 16)] = (h00 & 3) * 2
            low_b[s, pl.ds(1 * CH + g * 16, 16)] = (h10 & 3) * 2
            low_b[s, pl.ds(2 * CH + g * 16, 16)] = (h01 & 3) * 2
            low_b[s, pl.ds(3 * CH + g * 16, 16)] = (h11 & 3) * 2

    def fire(lvl):
        s = lvl % DEPTH
        return pltpu.async_copy(tab_hbm.at[idx_b.at[s]], rows_b.at[s], gsem)

    def combine(lvl):
        s = lvl % DEPTH

        @pl.loop(0, G)
        def _combine(g):
            sl = pl.ds(g * 16, 16)
            fx = fx_b[s, sl]
            fy = fy_b[s, sl]
            wx0 = 1.0 - fx
            wy0 = 1.0 - fy
            w00 = wx0 * wy0
            w10 = fx * wy0
            w01 = wx0 * fy
            w11 = fx * fy
            pt = g * 16 + iota
            rows = rows_b.at[s]
            c00 = low_b[s, pl.ds(0 * CH + g * 16, 16)]
            c10 = low_b[s, pl.ds(1 * CH + g * 16, 16)]
            c01 = low_b[s, pl.ds(2 * CH + g * 16, 16)]
            c11 = low_b[s, pl.ds(3 * CH + g * 16, 16)]
            e00a = plsc.load_gather(rows, [0 * CH + pt, c00])
            e00b = plsc.load_gather(rows, [0 * CH + pt, c00 + 1])
            e10a = plsc.load_gather(rows, [1 * CH + pt, c10])
            e10b = plsc.load_gather(rows, [1 * CH + pt, c10 + 1])
            e01a = plsc.load_gather(rows, [2 * CH + pt, c01])
            e01b = plsc.load_gather(rows, [2 * CH + pt, c01 + 1])
            e11a = plsc.load_gather(rows, [3 * CH + pt, c11])
            e11b = plsc.load_gather(rows, [3 * CH + pt, c11 + 1])
            feat0 = ((e00a * w00 + e10a * w10) + e01a * w01) + e11a * w11
            feat1 = ((e00b * w00 + e10b * w10) + e01b * w01) + e11b * w11
            out_b[2 * lvl, sl] = feat0
            out_b[2 * lvl + 1, sl] = feat1

    @pl.loop(0, NCHUNK)
    def _chunk(c):
        off = base + c * CH
        pltpu.sync_copy(u_hbm.at[pl.ds(off, CH)], u_v)
        pltpu.sync_copy(v_hbm.at[pl.ds(off, CH)], v_v)

        copies = {}
        for lvl in range(DEPTH):
            build(lvl)
            copies[lvl] = fire(lvl)
        for lvl in range(NUM_LEVELS):
            copies.pop(lvl).wait()
            combine(lvl)
            nxt = lvl + DEPTH
            if nxt < NUM_LEVELS:
                build(nxt)
                copies[nxt] = fire(nxt)

        bb = off // 65536
        pcol = off - bb * 65536
        obase = bb * (OUT_F * 65536) + pcol
        ocopies = [
            pltpu.async_copy(out_b.at[f],
                             out_hbm.at[pl.ds(obase + f * 65536, CH)], osem)
            for f in range(OUT_F)
        ]
        for d in ocopies:
            d.wait()


def kernel(coords, tables):
    b, n, _ = coords.shape
    u = coords[..., 1].reshape(N_POINTS)
    v = coords[..., 0].reshape(N_POINTS)
    tab1d = tables.transpose(0, 2, 1).reshape(TAB_F)
    packed = _pack(tab1d).reshape(NROWS, ROW_F)
    out = _encode(u, v, packed)
    return out.reshape(b, OUT_F, n).transpose(0, 2, 1)
